# baseline (speedup 1.0000x reference)
import functools

import jax
import jax.numpy as jnp
import numpy as np
from jax.experimental import pallas as pl
from jax.experimental.pallas import tpu as pltpu

K = 5
_C = 2048
_INTMAX = np.int32(0x7FFFFFFF)
_TINY = np.float32(np.finfo(np.float32).tiny)
_SPAN = np.float32(np.float32(1.0) - _TINY)

_K0 = np.uint32(0)
_K1 = np.uint32(42)
_K2 = np.uint32(0 ^ 42 ^ 0x1BD11BDA)
_ROT0 = (13, 15, 26, 6)
_ROT1 = (17, 29, 16, 24)


def _rotl(v, r):
    return (v << np.uint32(r)) | (v >> np.uint32(32 - r))


def _threefry_bits(cnt):
    ks = (_K0, _K1, _K2)
    x0 = jnp.full_like(cnt, ks[0])
    x1 = cnt + ks[1]
    for i in range(5):
        rots = _ROT0 if i % 2 == 0 else _ROT1
        for r in rots:
            x0 = x0 + x1
            x1 = _rotl(x1, r)
            x1 = x0 ^ x1
        x0 = x0 + ks[(i + 1) % 3]
        x1 = x1 + ks[(i + 2) % 3] + np.uint32(i + 1)
    return x0 ^ x1


def _gumbel_from_bits(bits):
    fb = (bits >> np.uint32(9)) | np.uint32(0x3F800000)
    f = jax.lax.bitcast_convert_type(fb, jnp.float32) - np.float32(1.0)
    u = jnp.maximum(_TINY, f * _SPAN + _TINY)
    return -jnp.log(-jnp.log(u))


def _table_body(*refs, V, C, B):
    i = pl.program_id(0)
    col = jax.lax.broadcasted_iota(jnp.int32, (B, C), 1) + i * C
    rowc = jax.lax.broadcasted_iota(jnp.int32, (B, C), 0) * V
    for k, ref in enumerate(refs):
        cnt = (rowc + col + k * B * V).astype(jnp.uint32)
        ref[0] = _gumbel_from_bits(_threefry_bits(cnt))


_TABLES = {}


def _gumbel_table(B, V):
    key = (B, V)
    tabs = _TABLES.get(key)
    if tabs is None:
        nb = pl.cdiv(V, _C)
        tabs = pl.pallas_call(
            functools.partial(_table_body, V=V, C=_C, B=B),
            grid=(nb,),
            in_specs=[],
            out_specs=[pl.BlockSpec((1, B, _C), lambda i: (i, 0, 0))
                       for _ in range(K)],
            out_shape=[jax.ShapeDtypeStruct((nb, B, _C), jnp.float32)
                       for _ in range(K)],
        )()
        tabs = jax.block_until_ready(tabs)
        _TABLES[key] = tabs
    return tabs


def _body(x_ref, t0_ref, t1_ref, t2_ref, t3_ref, t4_ref, sc_ref, ent_ref,
          samp_ref, m_ref, z_ref, a_ref, bv_ref, bi_ref, *, B, V, C):
    i = pl.program_id(0)
    nb = pl.num_programs(0)

    @pl.when(i == 0)
    def _init():
        m_ref[...] = jnp.full((B, 1), -jnp.inf, jnp.float32)
        z_ref[...] = jnp.zeros((B, 1), jnp.float32)
        a_ref[...] = jnp.zeros((B, 1), jnp.float32)
        bv_ref[...] = jnp.full((K, B, 1), -jnp.inf, jnp.float32)
        bi_ref[...] = jnp.zeros((K, B, 1), jnp.int32)

    s = x_ref[...]
    sc_ref[...] = s
    col = jax.lax.broadcasted_iota(jnp.int32, (B, C), 1) + i * C
    valid = col < V
    sneg = jnp.where(valid, s, -jnp.inf)
    sz = jnp.where(valid, s, 0.0)

    m_old = m_ref[...]
    m_new = jnp.maximum(m_old, jnp.max(sneg, axis=1, keepdims=True))
    scale = jnp.exp(m_old - m_new)
    t = jnp.exp(sneg - m_new)
    z_ref[...] = z_ref[...] * scale + jnp.sum(t, axis=1, keepdims=True)
    a_ref[...] = a_ref[...] * scale + jnp.sum(t * sz, axis=1, keepdims=True)
    m_ref[...] = m_new

    t_refs = (t0_ref, t1_ref, t2_ref, t3_ref, t4_ref)
    for k in range(K):
        cand = sneg + t_refs[k][0]
        cm = jnp.max(cand, axis=1, keepdims=True)
        idx = jnp.min(jnp.where(cand == cm, col, _INTMAX), axis=1,
                      keepdims=True)
        better = cm > bv_ref[k]
        bv_ref[k] = jnp.where(better, cm, bv_ref[k])
        bi_ref[k] = jnp.where(better, idx, bi_ref[k])

    @pl.when(i == nb - 1)
    def _finish():
        z = z_ref[...]
        ent_ref[...] = m_ref[...] + jnp.log(z) - a_ref[...] / z
        samp_ref[...] = bi_ref[...]


@jax.jit
def kernel(logits):
    B, V = logits.shape
    C = _C
    nb = pl.cdiv(V, C)
    tabs = _gumbel_table(B, V)
    scores, ent, samp = pl.pallas_call(
        functools.partial(_body, B=B, V=V, C=C),
        grid=(nb,),
        in_specs=[pl.BlockSpec((B, C), lambda i: (0, i))] + [
            pl.BlockSpec((1, B, C), lambda i: (i, 0, 0)) for _ in range(K)
        ],
        out_specs=[
            pl.BlockSpec((B, C), lambda i: (0, i)),
            pl.BlockSpec((B, 1), lambda i: (0, 0)),
            pl.BlockSpec((K, B, 1), lambda i: (0, 0, 0)),
        ],
        out_shape=[
            jax.ShapeDtypeStruct((B, V), jnp.float32),
            jax.ShapeDtypeStruct((B, 1), jnp.float32),
            jax.ShapeDtypeStruct((K, B, 1), jnp.int32),
        ],
        scratch_shapes=[
            pltpu.VMEM((B, 1), jnp.float32),
            pltpu.VMEM((B, 1), jnp.float32),
            pltpu.VMEM((B, 1), jnp.float32),
            pltpu.VMEM((K, B, 1), jnp.float32),
            pltpu.VMEM((K, B, 1), jnp.int32),
        ],
    )(logits, *tabs)
    return samp[..., 0], scores, ent[:, 0]


# speedup vs baseline: 5.5533x; 5.5533x over previous
import functools

import jax
import jax.numpy as jnp
import numpy as np
from jax.experimental import pallas as pl
from jax.experimental.pallas import tpu as pltpu

K = 5
_C = 2048
_INTMAX = np.int32(0x7FFFFFFF)
_TINY = np.float32(np.finfo(np.float32).tiny)
_SPAN = np.float32(np.float32(1.0) - _TINY)

_K0 = np.uint32(0)
_K1 = np.uint32(42)
_K2 = np.uint32(0 ^ 42 ^ 0x1BD11BDA)
_ROT0 = (13, 15, 26, 6)
_ROT1 = (17, 29, 16, 24)


def _rotl(v, r):
    return (v << np.uint32(r)) | (v >> np.uint32(32 - r))


def _threefry_bits(cnt):
    ks = (_K0, _K1, _K2)
    x0 = jnp.full_like(cnt, ks[0])
    x1 = cnt + ks[1]
    for i in range(5):
        rots = _ROT0 if i % 2 == 0 else _ROT1
        for r in rots:
            x0 = x0 + x1
            x1 = _rotl(x1, r)
            x1 = x0 ^ x1
        x0 = x0 + ks[(i + 1) % 3]
        x1 = x1 + ks[(i + 2) % 3] + np.uint32(i + 1)
    return x0 ^ x1


def _gumbel_from_bits(bits):
    fb = (bits >> np.uint32(9)) | np.uint32(0x3F800000)
    f = jax.lax.bitcast_convert_type(fb, jnp.float32) - np.float32(1.0)
    u = jnp.maximum(_TINY, f * _SPAN + _TINY)
    return -jnp.log(-jnp.log(u))


def _table_body(*refs, V, C, B):
    i = pl.program_id(0)
    col = jax.lax.broadcasted_iota(jnp.int32, (B, C), 1) + i * C
    rowc = jax.lax.broadcasted_iota(jnp.int32, (B, C), 0) * V
    for k, ref in enumerate(refs):
        cnt = (rowc + col + k * B * V).astype(jnp.uint32)
        ref[0] = _gumbel_from_bits(_threefry_bits(cnt))


_TABLES = {}


def _gumbel_table(B, V):
    key = (B, V)
    tabs = _TABLES.get(key)
    if tabs is None:
        nb = pl.cdiv(V, _C)
        builder = jax.jit(pl.pallas_call(
            functools.partial(_table_body, V=V, C=_C, B=B),
            grid=(nb,),
            in_specs=[],
            out_specs=[pl.BlockSpec((1, B, _C), lambda i: (i, 0, 0))
                       for _ in range(K)],
            out_shape=[jax.ShapeDtypeStruct((nb, B, _C), jnp.float32)
                       for _ in range(K)],
        ))
        with jax._src.core.eval_context():
            tabs = jax.block_until_ready(builder())
        _TABLES[key] = tabs
    return tabs


def _body(x_ref, t0_ref, t1_ref, t2_ref, t3_ref, t4_ref, sc_ref, ent_ref,
          samp_ref, m_ref, z_ref, a_ref, bv_ref, bi_ref, *, B, V, C):
    i = pl.program_id(0)
    nb = pl.num_programs(0)

    @pl.when(i == 0)
    def _init():
        m_ref[...] = jnp.full((B, 1), -jnp.inf, jnp.float32)
        z_ref[...] = jnp.zeros((B, 1), jnp.float32)
        a_ref[...] = jnp.zeros((B, 1), jnp.float32)
        bv_ref[...] = jnp.full((K, B, 1), -jnp.inf, jnp.float32)
        bi_ref[...] = jnp.zeros((K, B, 1), jnp.int32)

    s = x_ref[...]
    sc_ref[...] = s
    col = jax.lax.broadcasted_iota(jnp.int32, (B, C), 1) + i * C
    valid = col < V
    sneg = jnp.where(valid, s, -jnp.inf)
    sz = jnp.where(valid, s, 0.0)

    m_old = m_ref[...]
    m_new = jnp.maximum(m_old, jnp.max(sneg, axis=1, keepdims=True))
    scale = jnp.exp(m_old - m_new)
    t = jnp.exp(sneg - m_new)
    z_ref[...] = z_ref[...] * scale + jnp.sum(t, axis=1, keepdims=True)
    a_ref[...] = a_ref[...] * scale + jnp.sum(t * sz, axis=1, keepdims=True)
    m_ref[...] = m_new

    t_refs = (t0_ref, t1_ref, t2_ref, t3_ref, t4_ref)
    for k in range(K):
        cand = sneg + t_refs[k][0]
        cm = jnp.max(cand, axis=1, keepdims=True)
        idx = jnp.min(jnp.where(cand == cm, col, _INTMAX), axis=1,
                      keepdims=True)
        better = cm > bv_ref[k]
        bv_ref[k] = jnp.where(better, cm, bv_ref[k])
        bi_ref[k] = jnp.where(better, idx, bi_ref[k])

    @pl.when(i == nb - 1)
    def _finish():
        z = z_ref[...]
        ent_ref[...] = m_ref[...] + jnp.log(z) - a_ref[...] / z
        samp_ref[...] = bi_ref[...]


@jax.jit
def kernel(logits):
    B, V = logits.shape
    C = _C
    nb = pl.cdiv(V, C)
    tabs = _gumbel_table(B, V)
    scores, ent, samp = pl.pallas_call(
        functools.partial(_body, B=B, V=V, C=C),
        grid=(nb,),
        in_specs=[pl.BlockSpec((B, C), lambda i: (0, i))] + [
            pl.BlockSpec((1, B, C), lambda i: (i, 0, 0)) for _ in range(K)
        ],
        out_specs=[
            pl.BlockSpec((B, C), lambda i: (0, i)),
            pl.BlockSpec((B, 1), lambda i: (0, 0)),
            pl.BlockSpec((K, B, 1), lambda i: (0, 0, 0)),
        ],
        out_shape=[
            jax.ShapeDtypeStruct((B, V), jnp.float32),
            jax.ShapeDtypeStruct((B, 1), jnp.float32),
            jax.ShapeDtypeStruct((K, B, 1), jnp.int32),
        ],
        scratch_shapes=[
            pltpu.VMEM((B, 1), jnp.float32),
            pltpu.VMEM((B, 1), jnp.float32),
            pltpu.VMEM((B, 1), jnp.float32),
            pltpu.VMEM((K, B, 1), jnp.float32),
            pltpu.VMEM((K, B, 1), jnp.int32),
        ],
    )(logits, *tabs)
    return samp[..., 0], scores, ent[:, 0]
